# baseline (speedup 1.0000x reference)
import functools

import jax
import jax.numpy as jnp
from jax import lax
from jax.experimental import pallas as pl
from jax.experimental.pallas import tpu as pltpu
from jax.experimental.pallas import tpu_sc as plsc

N = 10000
E = 320000
D = 128
DH = D // 2
H = 512
L = 16
NC = 2
NS = 16
NW = NC * NS
NBUF = 4

EA = E // NS
CA = 80
NCA = EA // CA
EB = E // NW
CB = 64
NCB = EB // CB
CBT = EB - NCB * CB
NB2 = 3
NGS = 8

ZR = 624
ZR_LAST = N - ZR * (NS - 1)

_mesh = plsc.VectorSubcoreMesh(
    core_axis_name="c", subcore_axis_name="s", num_cores=NC, num_subcores=NS)
_BCAST_DNUMS = lax.GatherDimensionNumbers(
    offset_dims=(), collapsed_slice_dims=(0,), start_index_map=(0,))
_sc_params = pltpu.CompilerParams(needs_layout_passes=False,
                                  use_tc_tiling_on_sc=False)


def _f32(shape):
    return jax.ShapeDtypeStruct(shape, jnp.float32)


def _stripe_pair(sid, copy_fn):
    @pl.when(sid < NS - 1)
    def _():
        copy_fn(pl.ds(pl.multiple_of(sid * ZR, 8), ZR))

    @pl.when(sid == NS - 1)
    def _():
        copy_fn(pl.ds((NS - 1) * ZR, ZR_LAST))


@functools.partial(
    pl.kernel,
    out_type=_f32((NC, N, DH)),
    mesh=_mesh,
    scratch_types=[
        pltpu.VMEM((EA,), jnp.int32),
        pltpu.VMEM((NCA, CA), jnp.int32),
        pltpu.VMEM((NGS, CA, DH), jnp.float32),
        pltpu.VMEM_SHARED((N, DH), jnp.float32),
        [pltpu.SemaphoreType.DMA] * NGS,
        [pltpu.SemaphoreType.DMA] * NGS,
    ],
    compiler_params=_sc_params,
)
def _aggregate(h_hbm, src_hbm, dst3_hbm, z_hbm, p_hbm,
               sbuf, dbuf, rows, accum, gsems, ssems):
    cid = lax.axis_index("c")
    sid = lax.axis_index("s")
    hh = h_hbm.at[cid]
    base = sid * EA
    pltpu.sync_copy(src_hbm.at[pl.ds(pl.multiple_of(base, 8), EA)], sbuf)
    pltpu.sync_copy(dst3_hbm.at[sid], dbuf)
    _stripe_pair(sid, lambda rs: pltpu.sync_copy(z_hbm.at[rs], accum.at[rs]))
    plsc.subcore_barrier()

    def gather(k, s):
        pltpu.async_copy(
            hh.at[sbuf.at[pl.ds(k * CA, CA)]], rows.at[s], gsems[s])

    for j in range(NBUF):
        gather(j, j)

    def octet(kq, carry):
        for s in range(NGS):
            k = kq * NGS + s
            t = (s + NBUF) % NGS
            @pl.when(k < NCA)
            def _():
                pltpu.make_async_copy(
                    hh.at[sbuf.at[pl.ds(k * CA, CA)]], rows.at[s],
                    gsems[s]).wait()
                pltpu.async_copy(rows.at[s], accum.at[dbuf.at[k]], ssems[s],
                                 add=True)

                @pl.when(k + NBUF < NCA)
                def _():
                    @pl.when(k >= NBUF)
                    def _():
                        pltpu.make_async_copy(
                            rows.at[t],
                            accum.at[dbuf.at[k - NBUF]], ssems[t]).wait()
                    gather(k + NBUF, t)
        return carry

    lax.fori_loop(0, (NCA + NGS - 1) // NGS, octet, 0)
    for j in range(NGS):
        k = NCA - NGS + j
        s = k % NGS
        pltpu.make_async_copy(rows.at[s], accum.at[dbuf.at[k]],
                              ssems[s]).wait()
    plsc.subcore_barrier()
    _stripe_pair(sid, lambda rs: pltpu.sync_copy(accum.at[rs], p_hbm.at[cid].at[rs]))


@functools.partial(
    pl.kernel,
    out_type=_f32((E,)),
    mesh=_mesh,
    scratch_types=[
        pltpu.VMEM((EB,), jnp.int32),
        pltpu.VMEM((EB,), jnp.int32),
        pltpu.VMEM((NB2, CB, H), jnp.bfloat16),
        pltpu.VMEM((NB2, CB, H), jnp.bfloat16),
        pltpu.VMEM((CB, L), jnp.float32),
        pltpu.VMEM((NB2, CB), jnp.float32),
        pltpu.VMEM((H,), jnp.bfloat16),
        pltpu.VMEM((L,), jnp.float32),
        [pltpu.SemaphoreType.DMA] * NB2,
        [pltpu.SemaphoreType.DMA] * NB2,
    ],
    compiler_params=_sc_params,
)
def _edge_weights(g1_hbm, g2_hbm, src_hbm, dst_hbm, w2_hbm, b2_hbm, w_hbm,
                  cbuf, rbuf, g1b, g2b, accbuf, wbuf, w2v, b2v,
                  gsems, wsems):
    cid = lax.axis_index("c")
    sid = lax.axis_index("s")
    wid = cid * NS + sid
    base = wid * EB
    pltpu.sync_copy(w2_hbm, w2v)
    pltpu.sync_copy(b2_hbm, b2v)
    pltpu.sync_copy(src_hbm.at[pl.ds(pl.multiple_of(base, 8), EB)], cbuf)
    pltpu.sync_copy(dst_hbm.at[pl.ds(pl.multiple_of(base, 8), EB)], rbuf)
    lanes = lax.iota(jnp.int32, L)

    def fire(k, s, n):
        sl = pl.ds(k * CB, n)
        pltpu.async_copy(g1_hbm.at[cbuf.at[sl]],
                         g1b.at[s, pl.ds(0, n)], gsems[s])
        pltpu.async_copy(g2_hbm.at[rbuf.at[sl]],
                         g2b.at[s, pl.ds(0, n)], gsems[s])

    def drain(k, s, n):
        sl = pl.ds(k * CB, n)
        pltpu.make_async_copy(g1_hbm.at[cbuf.at[sl]],
                              g1b.at[s, pl.ds(0, n)], gsems[s]).wait()
        pltpu.make_async_copy(g2_hbm.at[rbuf.at[sl]],
                              g2b.at[s, pl.ds(0, n)], gsems[s]).wait()

    def wslice(k, n):
        return w_hbm.at[pl.ds(pl.multiple_of(base + k * CB, 8), n)]

    def compute(s, n):
        b2vec = b2v[...]

        def edge(e, carry2):
            accs = [jnp.zeros((L,), jnp.float32) for _ in range(4)]
            for kk in range(H // (2 * L)):
                csl = pl.ds(kk * 2 * L, 2 * L)
                hdn = jnp.maximum(g1b[s, e, csl] + g2b[s, e, csl],
                                  jnp.bfloat16(0.0))
                prod = hdn * w2v[csl]
                u0, u1 = plsc.unpack(prod, format=plsc.PackFormat.INTERLEAVED)
                accs[kk % 4] = accs[kk % 4] + (u0 + u1)
            accbuf[e, :] = (accs[0] + accs[1]) + (accs[2] + accs[3])
            return carry2

        lax.fori_loop(0, n, edge, 0)

        for g in range(n // L):
            wsums = [jnp.zeros((L,), jnp.float32) for _ in range(4)]
            rows_idx = lanes + (g * L)
            for l in range(L):
                cols_idx = jnp.full((L,), l, jnp.int32)
                wsums[l % 4] = wsums[l % 4] + plsc.load_gather(
                    accbuf, [rows_idx, cols_idx])
            wsum = (wsums[0] + wsums[1]) + (wsums[2] + wsums[3])
            wbuf[s, pl.ds(g * L, L)] = jnp.maximum(wsum + b2vec, 0.0)

    for j in range(NB2):
        fire(j, j, CB)

    def triple(kq, carry):
        for s in range(NB2):
            k = kq * NB2 + s
            drain(k, s, CB)
            @pl.when(k >= NB2)
            def _():
                pltpu.make_async_copy(wbuf.at[s], wslice(k - NB2, CB),
                                      wsems[s]).wait()
            compute(s, CB)
            pltpu.async_copy(wbuf.at[s], wslice(k, CB), wsems[s])

            @pl.when(k + NB2 < NCB)
            def _():
                fire(k + NB2, s, CB)
        return carry

    lax.fori_loop(0, NCB // NB2, triple, 0)
    for j in range(NB2):
        k = NCB - NB2 + j
        pltpu.make_async_copy(wbuf.at[k % NB2], wslice(k, CB),
                              wsems[k % NB2]).wait()
    fire(NCB, 0, CBT)
    drain(NCB, 0, CBT)
    compute(0, CBT)
    pltpu.sync_copy(wbuf.at[0, pl.ds(0, CBT)], wslice(NCB, CBT))


@functools.partial(
    pl.kernel,
    out_type=_f32((NC, N, DH)),
    mesh=_mesh,
    scratch_types=[
        pltpu.VMEM((EA,), jnp.int32),
        pltpu.VMEM((NCA, CA), jnp.int32),
        pltpu.VMEM((NBUF, CA), jnp.float32),
        pltpu.VMEM((NBUF, CA, DH), jnp.float32),
        pltpu.VMEM((NBUF, CA, DH), jnp.float32),
        pltpu.VMEM_SHARED((N, DH), jnp.float32),
        [pltpu.SemaphoreType.DMA] * NBUF,
        [pltpu.SemaphoreType.DMA] * NBUF,
    ],
    compiler_params=_sc_params,
)
def _weighted_aggregate(h_hbm, src_hbm, dst3_hbm, w_hbm, z_hbm, q_hbm,
                        sbuf, dbuf, wload, rows, mbuf, accum, gsems, ssems):
    cid = lax.axis_index("c")
    sid = lax.axis_index("s")
    hh = h_hbm.at[cid]
    base = sid * EA
    pltpu.sync_copy(src_hbm.at[pl.ds(pl.multiple_of(base, 8), EA)], sbuf)
    pltpu.sync_copy(dst3_hbm.at[sid], dbuf)
    _stripe_pair(sid, lambda rs: pltpu.sync_copy(z_hbm.at[rs], accum.at[rs]))
    plsc.subcore_barrier()

    def gather(k, s):
        off = pl.multiple_of(base + k * CA, 8)
        pltpu.async_copy(w_hbm.at[pl.ds(off, CA)], wload.at[s], gsems[s])
        pltpu.async_copy(
            hh.at[sbuf.at[pl.ds(k * CA, CA)]], rows.at[s], gsems[s])

    for j in range(NBUF):
        gather(j, j)

    def quad(kq, carry):
        for s in range(NBUF):
            k = kq * NBUF + s
            @pl.when(k < NCA)
            def _():
                off = pl.multiple_of(base + k * CA, 8)
                pltpu.make_async_copy(
                    w_hbm.at[pl.ds(off, CA)], wload.at[s], gsems[s]).wait()
                pltpu.make_async_copy(
                    hh.at[sbuf.at[pl.ds(k * CA, CA)]], rows.at[s],
                    gsems[s]).wait()

                @pl.when(k >= NBUF)
                def _():
                    pltpu.make_async_copy(
                        mbuf.at[s], accum.at[dbuf.at[k - NBUF]],
                        ssems[s]).wait()

                def msg(g, carry2):
                    wvec = wload[s, pl.ds(g * L, L)]
                    for ee in range(L):
                        wsc = lax.gather(
                            wvec, jnp.full((L, 1), ee, jnp.int32),
                            _BCAST_DNUMS, slice_sizes=(1,),
                            mode=lax.GatherScatterMode.PROMISE_IN_BOUNDS)
                        e = g * L + ee
                        for kk in range(DH // L):
                            csl = pl.ds(kk * L, L)
                            mbuf[s, e, csl] = rows[s, e, csl] * wsc
                    return carry2

                lax.fori_loop(0, CA // L, msg, 0)
                pltpu.async_copy(mbuf.at[s], accum.at[dbuf.at[k]], ssems[s],
                                 add=True)

                @pl.when(k + NBUF < NCA)
                def _():
                    gather(k + NBUF, s)
        return carry

    lax.fori_loop(0, (NCA + NBUF - 1) // NBUF, quad, 0)
    for j in range(NBUF):
        k = NCA - NBUF + j
        pltpu.make_async_copy(mbuf.at[k % NBUF], accum.at[dbuf.at[k]],
                              ssems[k % NBUF]).wait()
    plsc.subcore_barrier()
    _stripe_pair(sid, lambda rs: pltpu.sync_copy(accum.at[rs], q_hbm.at[cid].at[rs]))


def _h_body(x_ref, w_ref, o_ref):
    o_ref[0] = jnp.dot(x_ref[...], w_ref[...][:, :DH],
                       preferred_element_type=jnp.float32)
    o_ref[1] = jnp.dot(x_ref[...], w_ref[...][:, DH:],
                       preferred_element_type=jnp.float32)


def _tc_h(x, W):
    return pl.pallas_call(_h_body, out_shape=_f32((NC, N, DH)))(x, W)


BN = 2000


def _g_body(p_ref, w1a_ref, w1b_ref, b1_ref, g1_ref, g2_ref):
    emb = jnp.concatenate([p_ref[0], p_ref[1]], axis=1)
    g1_ref[...] = jnp.dot(
        emb, w1a_ref[...],
        preferred_element_type=jnp.float32).astype(jnp.bfloat16)
    g2_ref[...] = (jnp.dot(emb, w1b_ref[...], preferred_element_type=jnp.float32)
                   + b1_ref[...]).astype(jnp.bfloat16)


def _tc_g(p, w1a, w1b, b1):
    return pl.pallas_call(
        _g_body,
        grid=(N // BN,),
        in_specs=[
            pl.BlockSpec((NC, BN, DH), lambda i: (0, i, 0)),
            pl.BlockSpec((D, H), lambda i: (0, 0)),
            pl.BlockSpec((D, H), lambda i: (0, 0)),
            pl.BlockSpec((1, H), lambda i: (0, 0)),
        ],
        out_specs=[pl.BlockSpec((BN, H), lambda i: (i, 0)),
                   pl.BlockSpec((BN, H), lambda i: (i, 0))],
        out_shape=[jax.ShapeDtypeStruct((N, H), jnp.bfloat16),
                   jax.ShapeDtypeStruct((N, H), jnp.bfloat16)],
    )(p, w1a, w1b, b1)


def kernel(x, edge_index, W, W1, b1, W2, b2):
    src = edge_index[0]
    dst = edge_index[1]
    dst3 = dst.reshape(NS, NCA, CA)
    z = jnp.zeros((N, DH), jnp.float32)
    h = _tc_h(x, W)
    p = _aggregate(h, src, dst3, z)
    g1, g2 = _tc_g(p, W1[:D], W1[D:], b1.reshape(1, H))
    w = _edge_weights(g1, g2, src, dst, W2.reshape(H).astype(jnp.bfloat16),
                      jnp.broadcast_to(b2, (L,)))
    q = _weighted_aggregate(h, src, dst3, w, z)
    return jnp.concatenate([q[0], q[1]], axis=1)


# speedup vs baseline: 1.0142x; 1.0142x over previous
import functools

import jax
import jax.numpy as jnp
from jax import lax
from jax.experimental import pallas as pl
from jax.experimental.pallas import tpu as pltpu
from jax.experimental.pallas import tpu_sc as plsc

N = 10000
E = 320000
D = 128
DH = D // 2
H = 512
L = 16
NC = 2
NS = 16
NW = NC * NS
NBUF = 4

EA = E // NS
CA = 80
NCA = EA // CA
EB = E // NW
CB = 64
NCB = EB // CB
CBT = EB - NCB * CB
NB2 = 3
NGS = 8

ZR = 624
ZR_LAST = N - ZR * (NS - 1)

_mesh = plsc.VectorSubcoreMesh(
    core_axis_name="c", subcore_axis_name="s", num_cores=NC, num_subcores=NS)
_BCAST_DNUMS = lax.GatherDimensionNumbers(
    offset_dims=(), collapsed_slice_dims=(0,), start_index_map=(0,))
_sc_params = pltpu.CompilerParams(needs_layout_passes=False,
                                  use_tc_tiling_on_sc=False)


def _f32(shape):
    return jax.ShapeDtypeStruct(shape, jnp.float32)


def _stripe_pair(sid, copy_fn):
    @pl.when(sid < NS - 1)
    def _():
        copy_fn(pl.ds(pl.multiple_of(sid * ZR, 8), ZR))

    @pl.when(sid == NS - 1)
    def _():
        copy_fn(pl.ds((NS - 1) * ZR, ZR_LAST))


@functools.partial(
    pl.kernel,
    out_type=_f32((NC, N, DH)),
    mesh=_mesh,
    scratch_types=[
        pltpu.VMEM((EA,), jnp.int32),
        pltpu.VMEM((NCA, CA), jnp.int32),
        pltpu.VMEM((NGS, CA, DH), jnp.float32),
        pltpu.VMEM_SHARED((N, DH), jnp.float32),
        [pltpu.SemaphoreType.DMA] * NGS,
        [pltpu.SemaphoreType.DMA] * NGS,
    ],
    compiler_params=_sc_params,
)
def _aggregate(h_hbm, src_hbm, dst3_hbm, z_hbm, p_hbm,
               sbuf, dbuf, rows, accum, gsems, ssems):
    cid = lax.axis_index("c")
    sid = lax.axis_index("s")
    hh = h_hbm.at[cid]
    base = sid * EA
    pltpu.sync_copy(src_hbm.at[pl.ds(pl.multiple_of(base, 8), EA)], sbuf)
    pltpu.sync_copy(dst3_hbm.at[sid], dbuf)
    _stripe_pair(sid, lambda rs: pltpu.sync_copy(z_hbm.at[rs], accum.at[rs]))
    plsc.subcore_barrier()

    def gather(k, s):
        pltpu.async_copy(
            hh.at[sbuf.at[pl.ds(k * CA, CA)]], rows.at[s], gsems[s])

    for j in range(NBUF):
        gather(j, j)

    def octet(kq, carry):
        for s in range(NGS):
            k = kq * NGS + s
            t = (s + NBUF) % NGS
            @pl.when(k < NCA)
            def _():
                pltpu.make_async_copy(
                    hh.at[sbuf.at[pl.ds(k * CA, CA)]], rows.at[s],
                    gsems[s]).wait()
                pltpu.async_copy(rows.at[s], accum.at[dbuf.at[k]], ssems[s],
                                 add=True)

                @pl.when(k + NBUF < NCA)
                def _():
                    @pl.when(k >= NBUF)
                    def _():
                        pltpu.make_async_copy(
                            rows.at[t],
                            accum.at[dbuf.at[k - NBUF]], ssems[t]).wait()
                    gather(k + NBUF, t)
        return carry

    lax.fori_loop(0, (NCA + NGS - 1) // NGS, octet, 0)
    for j in range(NGS):
        k = NCA - NGS + j
        s = k % NGS
        pltpu.make_async_copy(rows.at[s], accum.at[dbuf.at[k]],
                              ssems[s]).wait()
    plsc.subcore_barrier()
    _stripe_pair(sid, lambda rs: pltpu.sync_copy(accum.at[rs], p_hbm.at[cid].at[rs]))


@functools.partial(
    pl.kernel,
    out_type=_f32((E,)),
    mesh=_mesh,
    scratch_types=[
        pltpu.VMEM((EB,), jnp.int32),
        pltpu.VMEM((EB,), jnp.int32),
        pltpu.VMEM((NB2, CB, H), jnp.bfloat16),
        pltpu.VMEM((NB2, CB, H), jnp.bfloat16),
        pltpu.VMEM((CB, L), jnp.float32),
        pltpu.VMEM((NB2, CB), jnp.float32),
        pltpu.VMEM((H,), jnp.bfloat16),
        pltpu.VMEM((L,), jnp.float32),
        [pltpu.SemaphoreType.DMA] * NB2,
        [pltpu.SemaphoreType.DMA] * NB2,
    ],
    compiler_params=_sc_params,
)
def _edge_weights(g1_hbm, g2_hbm, src_hbm, dst_hbm, w2_hbm, b2_hbm, w_hbm,
                  cbuf, rbuf, g1b, g2b, accbuf, wbuf, w2v, b2v,
                  gsems, wsems):
    cid = lax.axis_index("c")
    sid = lax.axis_index("s")
    wid = cid * NS + sid
    base = wid * EB
    pltpu.sync_copy(w2_hbm, w2v)
    pltpu.sync_copy(b2_hbm, b2v)
    pltpu.sync_copy(src_hbm.at[pl.ds(pl.multiple_of(base, 8), EB)], cbuf)
    pltpu.sync_copy(dst_hbm.at[pl.ds(pl.multiple_of(base, 8), EB)], rbuf)
    lanes = lax.iota(jnp.int32, L)

    def fire(k, s, n):
        sl = pl.ds(k * CB, n)
        pltpu.async_copy(g1_hbm.at[cbuf.at[sl]],
                         g1b.at[s, pl.ds(0, n)], gsems[s])
        pltpu.async_copy(g2_hbm.at[rbuf.at[sl]],
                         g2b.at[s, pl.ds(0, n)], gsems[s])

    def drain(k, s, n):
        sl = pl.ds(k * CB, n)
        pltpu.make_async_copy(g1_hbm.at[cbuf.at[sl]],
                              g1b.at[s, pl.ds(0, n)], gsems[s]).wait()
        pltpu.make_async_copy(g2_hbm.at[rbuf.at[sl]],
                              g2b.at[s, pl.ds(0, n)], gsems[s]).wait()

    def wslice(k, n):
        return w_hbm.at[pl.ds(pl.multiple_of(base + k * CB, 8), n)]

    def compute(s, n):
        b2vec = b2v[...]

        def edge(q, carry2):
            for u in range(2):
                e = q * 2 + u
                accs = [jnp.zeros((L,), jnp.float32) for _ in range(4)]
                for kk in range(H // (2 * L)):
                    csl = pl.ds(kk * 2 * L, 2 * L)
                    hdn = jnp.maximum(g1b[s, e, csl] + g2b[s, e, csl],
                                      jnp.bfloat16(0.0))
                    prod = hdn * w2v[csl]
                    u0, u1 = plsc.unpack(
                        prod, format=plsc.PackFormat.INTERLEAVED)
                    accs[kk % 4] = accs[kk % 4] + (u0 + u1)
                accbuf[e, :] = (accs[0] + accs[1]) + (accs[2] + accs[3])
            return carry2

        lax.fori_loop(0, n // 2, edge, 0)

        for g in range(n // L):
            wsums = [jnp.zeros((L,), jnp.float32) for _ in range(4)]
            rows_idx = lanes + (g * L)
            for l in range(L):
                cols_idx = jnp.full((L,), l, jnp.int32)
                wsums[l % 4] = wsums[l % 4] + plsc.load_gather(
                    accbuf, [rows_idx, cols_idx])
            wsum = (wsums[0] + wsums[1]) + (wsums[2] + wsums[3])
            wbuf[s, pl.ds(g * L, L)] = jnp.maximum(wsum + b2vec, 0.0)

    for j in range(NB2):
        fire(j, j, CB)

    def triple(kq, carry):
        for s in range(NB2):
            k = kq * NB2 + s
            drain(k, s, CB)
            @pl.when(k >= NB2)
            def _():
                pltpu.make_async_copy(wbuf.at[s], wslice(k - NB2, CB),
                                      wsems[s]).wait()
            compute(s, CB)
            pltpu.async_copy(wbuf.at[s], wslice(k, CB), wsems[s])

            @pl.when(k + NB2 < NCB)
            def _():
                fire(k + NB2, s, CB)
        return carry

    lax.fori_loop(0, NCB // NB2, triple, 0)
    for j in range(NB2):
        k = NCB - NB2 + j
        pltpu.make_async_copy(wbuf.at[k % NB2], wslice(k, CB),
                              wsems[k % NB2]).wait()
    fire(NCB, 0, CBT)
    drain(NCB, 0, CBT)
    compute(0, CBT)
    pltpu.sync_copy(wbuf.at[0, pl.ds(0, CBT)], wslice(NCB, CBT))


@functools.partial(
    pl.kernel,
    out_type=_f32((NC, N, DH)),
    mesh=_mesh,
    scratch_types=[
        pltpu.VMEM((EA,), jnp.int32),
        pltpu.VMEM((NCA, CA), jnp.int32),
        pltpu.VMEM((NBUF, CA), jnp.float32),
        pltpu.VMEM((NBUF, CA, DH), jnp.float32),
        pltpu.VMEM((NBUF, CA, DH), jnp.float32),
        pltpu.VMEM_SHARED((N, DH), jnp.float32),
        [pltpu.SemaphoreType.DMA] * NBUF,
        [pltpu.SemaphoreType.DMA] * NBUF,
    ],
    compiler_params=_sc_params,
)
def _weighted_aggregate(h_hbm, src_hbm, dst3_hbm, w_hbm, z_hbm, q_hbm,
                        sbuf, dbuf, wload, rows, mbuf, accum, gsems, ssems):
    cid = lax.axis_index("c")
    sid = lax.axis_index("s")
    hh = h_hbm.at[cid]
    base = sid * EA
    pltpu.sync_copy(src_hbm.at[pl.ds(pl.multiple_of(base, 8), EA)], sbuf)
    pltpu.sync_copy(dst3_hbm.at[sid], dbuf)
    _stripe_pair(sid, lambda rs: pltpu.sync_copy(z_hbm.at[rs], accum.at[rs]))
    plsc.subcore_barrier()

    def gather(k, s):
        off = pl.multiple_of(base + k * CA, 8)
        pltpu.async_copy(w_hbm.at[pl.ds(off, CA)], wload.at[s], gsems[s])
        pltpu.async_copy(
            hh.at[sbuf.at[pl.ds(k * CA, CA)]], rows.at[s], gsems[s])

    for j in range(NBUF):
        gather(j, j)

    def quad(kq, carry):
        for s in range(NBUF):
            k = kq * NBUF + s
            @pl.when(k < NCA)
            def _():
                off = pl.multiple_of(base + k * CA, 8)
                pltpu.make_async_copy(
                    w_hbm.at[pl.ds(off, CA)], wload.at[s], gsems[s]).wait()
                pltpu.make_async_copy(
                    hh.at[sbuf.at[pl.ds(k * CA, CA)]], rows.at[s],
                    gsems[s]).wait()

                @pl.when(k >= NBUF)
                def _():
                    pltpu.make_async_copy(
                        mbuf.at[s], accum.at[dbuf.at[k - NBUF]],
                        ssems[s]).wait()

                def msg(g, carry2):
                    wvec = wload[s, pl.ds(g * L, L)]
                    for ee in range(L):
                        wsc = lax.gather(
                            wvec, jnp.full((L, 1), ee, jnp.int32),
                            _BCAST_DNUMS, slice_sizes=(1,),
                            mode=lax.GatherScatterMode.PROMISE_IN_BOUNDS)
                        e = g * L + ee
                        for kk in range(DH // L):
                            csl = pl.ds(kk * L, L)
                            mbuf[s, e, csl] = rows[s, e, csl] * wsc
                    return carry2

                lax.fori_loop(0, CA // L, msg, 0)
                pltpu.async_copy(mbuf.at[s], accum.at[dbuf.at[k]], ssems[s],
                                 add=True)

                @pl.when(k + NBUF < NCA)
                def _():
                    gather(k + NBUF, s)
        return carry

    lax.fori_loop(0, (NCA + NBUF - 1) // NBUF, quad, 0)
    for j in range(NBUF):
        k = NCA - NBUF + j
        pltpu.make_async_copy(mbuf.at[k % NBUF], accum.at[dbuf.at[k]],
                              ssems[k % NBUF]).wait()
    plsc.subcore_barrier()
    _stripe_pair(sid, lambda rs: pltpu.sync_copy(accum.at[rs], q_hbm.at[cid].at[rs]))


def _h_body(x_ref, w_ref, o_ref):
    o_ref[0] = jnp.dot(x_ref[...], w_ref[...][:, :DH],
                       preferred_element_type=jnp.float32)
    o_ref[1] = jnp.dot(x_ref[...], w_ref[...][:, DH:],
                       preferred_element_type=jnp.float32)


def _tc_h(x, W):
    return pl.pallas_call(_h_body, out_shape=_f32((NC, N, DH)))(x, W)


BN = 2000


def _g_body(p_ref, w1a_ref, w1b_ref, b1_ref, g1_ref, g2_ref):
    emb = jnp.concatenate([p_ref[0], p_ref[1]], axis=1)
    g1_ref[...] = jnp.dot(
        emb, w1a_ref[...],
        preferred_element_type=jnp.float32).astype(jnp.bfloat16)
    g2_ref[...] = (jnp.dot(emb, w1b_ref[...], preferred_element_type=jnp.float32)
                   + b1_ref[...]).astype(jnp.bfloat16)


def _tc_g(p, w1a, w1b, b1):
    return pl.pallas_call(
        _g_body,
        grid=(N // BN,),
        in_specs=[
            pl.BlockSpec((NC, BN, DH), lambda i: (0, i, 0)),
            pl.BlockSpec((D, H), lambda i: (0, 0)),
            pl.BlockSpec((D, H), lambda i: (0, 0)),
            pl.BlockSpec((1, H), lambda i: (0, 0)),
        ],
        out_specs=[pl.BlockSpec((BN, H), lambda i: (i, 0)),
                   pl.BlockSpec((BN, H), lambda i: (i, 0))],
        out_shape=[jax.ShapeDtypeStruct((N, H), jnp.bfloat16),
                   jax.ShapeDtypeStruct((N, H), jnp.bfloat16)],
    )(p, w1a, w1b, b1)


def kernel(x, edge_index, W, W1, b1, W2, b2):
    src = edge_index[0]
    dst = edge_index[1]
    dst3 = dst.reshape(NS, NCA, CA)
    z = jnp.zeros((N, DH), jnp.float32)
    h = _tc_h(x, W)
    p = _aggregate(h, src, dst3, z)
    g1, g2 = _tc_g(p, W1[:D], W1[D:], b1.reshape(1, H))
    w = _edge_weights(g1, g2, src, dst, W2.reshape(H).astype(jnp.bfloat16),
                      jnp.broadcast_to(b2, (L,)))
    q = _weighted_aggregate(h, src, dst3, w, z)
    return jnp.concatenate([q[0], q[1]], axis=1)
